# baseline (speedup 1.0000x reference)
import functools

import jax
import jax.numpy as jnp
from jax.experimental import pallas as pl
from jax.experimental.pallas import tpu as pltpu
from jax.experimental.pallas import tpu_sc as plsc


def kernel(x, table):
    row = x.shape[1]
    emb = table.shape[1]

    mesh = plsc.ScalarSubcoreMesh(axis_name="c", num_cores=1)

    @functools.partial(
        pl.kernel,
        mesh=mesh,
        out_type=jax.ShapeDtypeStruct((1, emb), table.dtype),
    )
    def lookup(table_hbm, out_hbm):
        pltpu.sync_copy(table_hbm.at[pl.ds(row, 1), :], out_hbm)

    return lookup(table).reshape((emb,))


# speedup vs baseline: 1.0045x; 1.0045x over previous
import functools

import jax
from jax.experimental import pallas as pl
from jax.experimental.pallas import tpu as pltpu
from jax.experimental.pallas import tpu_sc as plsc


def kernel(x, table):
    row = x.shape[1]
    emb = table.shape[1]

    mesh = plsc.ScalarSubcoreMesh(axis_name="c", num_cores=1)

    @functools.partial(
        pl.kernel,
        mesh=mesh,
        out_type=jax.ShapeDtypeStruct((1, emb), table.dtype),
    )
    def lookup(table_hbm, out_hbm):
        pltpu.sync_copy(table_hbm.at[pl.ds(row, 1), :], out_hbm)

    return lookup(table).reshape((emb,))
